# baseline (speedup 1.0000x reference)
import jax
import jax.numpy as jnp
from jax.experimental import pallas as pl


def _gat(x, src, dst, W, a_s, a_d, n_nodes):
    Hh, F = a_s.shape
    Wh = (x @ W).reshape(-1, Hh, F)
    alpha_s = jnp.sum(Wh * a_s[None, :, :], axis=-1)
    alpha_d = jnp.sum(Wh * a_d[None, :, :], axis=-1)
    e = jax.nn.leaky_relu(alpha_s[src] + alpha_d[dst], 0.2)
    emax = jax.ops.segment_max(e, dst, num_segments=n_nodes)
    emax = jnp.where(jnp.isfinite(emax), emax, 0.0)
    m = jnp.exp(e - emax[dst])
    denom = jax.ops.segment_sum(m, dst, num_segments=n_nodes)
    coef = m / (denom[dst] + 1e-16)
    out = jax.ops.segment_sum(coef[:, :, None] * Wh[src], dst, num_segments=n_nodes)
    return jax.nn.elu(out.reshape(out.shape[0], Hh * F))


def _tail_kernel(x_ref, wd_ref, bd_ref, o_ref):
    x = x_ref[...]
    nrm = jnp.maximum(jnp.sqrt(jnp.sum(x * x)), 1e-12)
    o_ref[...] = (jnp.sum(x * wd_ref[...], axis=1, keepdims=True) / nrm
                  + bd_ref[...])


def kernel(node_feats, edge_index_int, edge_index_nh, W1, a1_src, a1_dst,
           W2, a2_src, a2_dst, Wd, bd):
    n = node_feats.shape[0]
    src_i, dst_i = edge_index_int[0], edge_index_int[1]
    src_n, dst_n = edge_index_nh[0], edge_index_nh[1]
    h_int = _gat(node_feats, src_i, dst_i, W1, a1_src, a1_dst, n)
    h_nh = _gat(node_feats, src_n, dst_n, W1, a1_src, a1_dst, n)
    h_int = _gat(h_int, src_i, dst_i, W2, a2_src, a2_dst, n)
    h_nh = _gat(h_nh, src_n, dst_n, W2, a2_src, a2_dst, n)
    x = jnp.concatenate((h_int, h_nh), axis=1)
    x = jnp.sum(x, 0).reshape(1, -1)
    out = pl.pallas_call(
        _tail_kernel,
        out_shape=jax.ShapeDtypeStruct((1, 1), jnp.float32),
    )(x, Wd.reshape(1, -1), bd.reshape(1, 1))
    return jnp.squeeze(out, 1)


# speedup vs baseline: 82.3435x; 82.3435x over previous
import functools

import jax
import jax.numpy as jnp
from jax import lax
from jax.experimental import pallas as pl
from jax.experimental.pallas import tpu as pltpu
from jax.experimental.pallas import tpu_sc as plsc

N = 50000
E = 800000
NP = 53248
BSZ = 4096
NBKT = 13
BSH = 12
NW = 32
EPW = 25600
EPAD = NW * EPW
RCAP = 28800
CH = 128
NHEAD = 6
D1, D2 = 96, 192
ROWBLK = 512


def _splat(s):
    return lax.broadcast_in_dim(s, (16,), ())


@functools.lru_cache(maxsize=None)
def _sc_mesh():
    return plsc.VectorSubcoreMesh(core_axis_name="c", subcore_axis_name="s")


_SC_PARAMS = pltpu.CompilerParams(needs_layout_passes=False,
                                  use_tc_tiling_on_sc=False)


def _bin_call(src_pad, dst_pad):
    @functools.partial(
        pl.kernel,
        out_type=(
            jax.ShapeDtypeStruct((NBKT + 1, NW * RCAP), jnp.int32),
            jax.ShapeDtypeStruct((NBKT + 1, NW * RCAP), jnp.int32),
            jax.ShapeDtypeStruct((NW, 16), jnp.int32),
        ),
        mesh=_sc_mesh(),
        compiler_params=_SC_PARAMS,
        scratch_types=dict(
            srcv=pltpu.VMEM((EPW,), jnp.int32),
            dstv=pltpu.VMEM((EPW,), jnp.int32),
            lsrc=pltpu.VMEM((RCAP,), jnp.int32),
            ldst=pltpu.VMEM((RCAP,), jnp.int32),
            cntv=pltpu.VMEM((16,), jnp.int32),
        ),
    )
    def bink(src_h, dst_h, bsrc_h, bdst_h, cnt_h, srcv, dstv, lsrc, ldst,
             cntv):
        cid = lax.axis_index("c")
        sid = lax.axis_index("s")
        w = sid * 2 + cid
        lane = lax.iota(jnp.int32, 16)
        base = w * EPW
        for c in range(8):
            pltpu.sync_copy(src_h.at[pl.ds(base + c * 3200, 3200)],
                            srcv.at[pl.ds(c * 3200, 3200)])
            pltpu.sync_copy(dst_h.at[pl.ds(base + c * 3200, 3200)],
                            dstv.at[pl.ds(c * 3200, 3200)])
        cvec = jnp.zeros((16,), jnp.int32)
        for b in range(NBKT):
            def grp(g, cur, b=b):
                dv = dstv[pl.ds(g * 16, 16)]
                sv = srcv[pl.ds(g * 16, 16)]
                mask = (dv >> BSH) == b
                sk, svv, _ = plsc.sort_key_val(dv, sv, mask=mask)
                ldst[pl.ds(cur, 16)] = sk
                lsrc[pl.ds(cur, 16)] = svv
                pc = plsc.all_reduce_population_count(mask)
                return cur + pc[0]

            cursor = lax.fori_loop(0, EPW // 16, grp, jnp.int32(0))
            pdst = _splat(jnp.int32(b * BSZ)) + lane
            psrc = lane
            for pg in range(8):
                ldst[pl.ds(cursor + pg * 16, 16)] = pdst
                lsrc[pl.ds(cursor + pg * 16, 16)] = psrc
            rounded = ((cursor + 127) >> 7) << 7
            trips = (rounded + 2047) >> 11
            rbase = w * RCAP

            def dout(c, _, b=b):
                pltpu.sync_copy(lsrc.at[pl.ds(c * 2048, 2048)],
                                bsrc_h.at[b, pl.ds(rbase + c * 2048, 2048)])
                pltpu.sync_copy(ldst.at[pl.ds(c * 2048, 2048)],
                                bdst_h.at[b, pl.ds(rbase + c * 2048, 2048)])
                return 0

            lax.fori_loop(0, trips, dout, 0)
            cvec = jnp.where(lane == b, _splat(cursor), cvec)
        cntv[...] = cvec
        pltpu.sync_copy(cntv, cnt_h.at[w])

    return bink(src_pad, dst_pad)


def _edge_call(D, bsrc, bdst, cnt, wh, a_s, a_d, u16):
    nvreg = D // 16
    fdim = D // NHEAD
    head_of = [(k * 16) // fdim for k in range(nvreg)]

    @functools.partial(
        pl.kernel,
        out_type=jax.ShapeDtypeStruct((NP, D), jnp.float32),
        mesh=_sc_mesh(),
        compiler_params=_SC_PARAMS,
        scratch_types=dict(
            cntv=pltpu.VMEM((NW, 16), jnp.int32),
            uv=pltpu.VMEM((16,), jnp.float32),
            srcb=pltpu.VMEM((CH,), jnp.int32),
            dstb=pltpu.VMEM((CH,), jnp.int32),
            locb=pltpu.VMEM((CH,), jnp.int32),
            asb=pltpu.VMEM((CH, 16), jnp.float32),
            adb=pltpu.VMEM((CH, 16), jnp.float32),
            mb=pltpu.VMEM((CH, 16), jnp.float32),
            rows=pltpu.VMEM((CH, D), jnp.float32),
            zbuf=pltpu.VMEM((64, D), jnp.float32),
            zden=pltpu.VMEM((64, 16), jnp.float32),
            fb=pltpu.VMEM((64, D), jnp.float32),
            dnb=pltpu.VMEM((64, 16), jnp.float32),
            acc_sh=pltpu.VMEM_SHARED((BSZ, D), jnp.float32),
            den_sh=pltpu.VMEM_SHARED((BSZ, 16), jnp.float32),
            sem1=pltpu.SemaphoreType.DMA,
            sem2=pltpu.SemaphoreType.DMA,
            sem3=pltpu.SemaphoreType.DMA,
        ),
    )
    def edgek(bsrc_h, bdst_h, cnt_h, wh_h, as_h, ad_h, u_h, hout_h,
              cntv, uv, srcb, dstb, locb, asb, adb, mb, rows, zbuf, zden,
              fb, dnb, acc_sh, den_sh, sem1, sem2, sem3):
        cid = lax.axis_index("c")
        sid = lax.axis_index("s")
        lane = lax.iota(jnp.int32, 16)
        zero16 = jnp.zeros((16,), jnp.float32)
        hidx = [jnp.full((16,), h, jnp.int32) for h in range(NHEAD)]
        pltpu.sync_copy(cnt_h, cntv)
        pltpu.sync_copy(u_h, uv)
        uvv = uv[pl.ds(0, 16)]

        def zrow(r, _):
            for k in range(nvreg):
                zbuf[r, pl.ds(k * 16, 16)] = zero16
            zden[r, pl.ds(0, 16)] = zero16
            return 0

        lax.fori_loop(0, 64, zrow, 0)

        def slot_body(slot, _):
            b = slot * 2 + cid

            @pl.when(b < NBKT)
            def _process():
                def zc(i, _):
                    pltpu.sync_copy(zbuf, acc_sh.at[pl.ds(sid * 256 + i * 64,
                                                          64)])
                    pltpu.sync_copy(zden, den_sh.at[pl.ds(sid * 256 + i * 64,
                                                          64)])
                    return 0

                lax.fori_loop(0, 4, zc, 0)
                plsc.subcore_barrier()

                for wo in range(2):
                    w = sid * 2 + wo
                    crow = cntv[w, pl.ds(0, 16)]
                    cnt_wb = jnp.take(crow, _splat(b))[0]
                    trips = (cnt_wb + (CH - 1)) >> 7

                    def chunk(c, _, b=b, w=w, cnt_wb=cnt_wb):
                        off = w * RCAP + c * CH
                        pltpu.sync_copy(bsrc_h.at[b, pl.ds(off, CH)], srcb)
                        pltpu.sync_copy(bdst_h.at[b, pl.ds(off, CH)], dstb)
                        cp1 = pltpu.async_copy(as_h.at[srcb], asb, sem1)
                        cp2 = pltpu.async_copy(ad_h.at[dstb], adb, sem2)
                        cp3 = pltpu.async_copy(wh_h.at[srcb], rows, sem3)

                        def lg(g, _):
                            dv = dstb[pl.ds(g * 16, 16)]
                            locb[pl.ds(g * 16, 16)] = dv & (BSZ - 1)
                            return 0

                        lax.fori_loop(0, CH // 16, lg, 0)
                        cp1.wait()
                        cp2.wait()
                        ebase = c * CH

                        def edge_m(e, _):
                            asv = asb[e, pl.ds(0, 16)]
                            adv = adb[e, pl.ds(0, 16)]
                            s = asv + adv
                            ev = jnp.maximum(s, 0.2 * s) - uvv
                            m = jnp.exp(ev)
                            valid = _splat(ebase + e) < _splat(cnt_wb)
                            m = jnp.where(valid, m, 0.0)
                            mb[e, pl.ds(0, 16)] = m
                            return 0

                        lax.fori_loop(0, CH, edge_m, 0)
                        cp3.wait()

                        def edge_scale(e, _):
                            m = mb[e, pl.ds(0, 16)]
                            sps = [jnp.take(m, hidx[h]) for h in range(NHEAD)]
                            for k in range(nvreg):
                                r = rows[e, pl.ds(k * 16, 16)]
                                rows[e, pl.ds(k * 16, 16)] = r * sps[head_of[k]]
                            return 0

                        lax.fori_loop(0, CH, edge_scale, 0)
                        pltpu.sync_copy(mb, den_sh.at[locb], add=True)
                        pltpu.sync_copy(rows, acc_sh.at[locb], add=True)
                        return 0

                    lax.fori_loop(0, trips, chunk, 0)
                plsc.subcore_barrier()

                def fin(i, _):
                    r0 = sid * 256 + i * 64
                    pltpu.sync_copy(acc_sh.at[pl.ds(r0, 64)], fb)
                    pltpu.sync_copy(den_sh.at[pl.ds(r0, 64)], dnb)

                    def frow(n, _):
                        dv = dnb[n, pl.ds(0, 16)]
                        rec = 1.0 / (dv + 1e-16)
                        sps = [jnp.take(rec, hidx[h]) for h in range(NHEAD)]
                        for k in range(nvreg):
                            x = fb[n, pl.ds(k * 16, 16)] * sps[head_of[k]]
                            y = jnp.where(x > 0, x, jnp.exp(x) - 1.0)
                            fb[n, pl.ds(k * 16, 16)] = y
                        return 0

                    lax.fori_loop(0, 64, frow, 0)
                    pltpu.sync_copy(fb, hout_h.at[pl.ds(b * BSZ + r0, 64)])
                    return 0

                lax.fori_loop(0, 4, fin, 0)
                plsc.subcore_barrier()

            return 0

        lax.fori_loop(0, (NBKT + 1) // 2, slot_body, 0)

    return edgek(bsrc, bdst, cnt, wh, a_s, a_d, u16)


def _prep_call(x_pad, w_pad, ase, ade, D):
    kdim = x_pad.shape[1]
    grid = NP // ROWBLK

    def prep(x_ref, w_ref, ase_ref, ade_ref, wh_ref, as_ref, ad_ref):
        xb = x_ref[...]
        whb = jnp.dot(xb, w_ref[...], preferred_element_type=jnp.float32)
        wh_ref[...] = whb
        as_ref[...] = jnp.dot(whb, ase_ref[...],
                              preferred_element_type=jnp.float32)
        ad_ref[...] = jnp.dot(whb, ade_ref[...],
                              preferred_element_type=jnp.float32)

    return pl.pallas_call(
        prep,
        grid=(grid,),
        in_specs=[
            pl.BlockSpec((ROWBLK, kdim), lambda i: (i, 0)),
            pl.BlockSpec((kdim, D), lambda i: (0, 0)),
            pl.BlockSpec((D, 16), lambda i: (0, 0)),
            pl.BlockSpec((D, 16), lambda i: (0, 0)),
        ],
        out_specs=[
            pl.BlockSpec((ROWBLK, D), lambda i: (i, 0)),
            pl.BlockSpec((ROWBLK, 16), lambda i: (i, 0)),
            pl.BlockSpec((ROWBLK, 16), lambda i: (i, 0)),
        ],
        out_shape=[
            jax.ShapeDtypeStruct((NP, D), jnp.float32),
            jax.ShapeDtypeStruct((NP, 16), jnp.float32),
            jax.ShapeDtypeStruct((NP, 16), jnp.float32),
        ],
    )(x_pad, w_pad, ase, ade)


def _u_call(a_s, a_d):
    def uk(as_ref, ad_ref, u_ref):
        u = (jnp.max(as_ref[...], axis=0, keepdims=True)
             + jnp.max(ad_ref[...], axis=0, keepdims=True))
        u_ref[...] = jnp.maximum(u, 0.2 * u)

    return pl.pallas_call(
        uk,
        out_shape=jax.ShapeDtypeStruct((1, 16), jnp.float32),
    )(a_s, a_d)


def _pool_call(h_i, h_n):
    grid = NP // ROWBLK

    def poolk(hi_ref, hn_ref, o_ref):
        @pl.when(pl.program_id(0) == 0)
        def _init():
            o_ref[...] = jnp.zeros_like(o_ref)

        s1 = jnp.sum(hi_ref[...], axis=0, keepdims=True)
        s2 = jnp.sum(hn_ref[...], axis=0, keepdims=True)
        o_ref[...] += jnp.concatenate([s1, s2], axis=1)

    return pl.pallas_call(
        poolk,
        grid=(grid,),
        in_specs=[
            pl.BlockSpec((ROWBLK, D2), lambda i: (i, 0)),
            pl.BlockSpec((ROWBLK, D2), lambda i: (i, 0)),
        ],
        out_specs=pl.BlockSpec((1, 2 * D2), lambda i: (0, 0)),
        out_shape=jax.ShapeDtypeStruct((1, 2 * D2), jnp.float32),
    )(h_i, h_n)


def _tail_call(pooled, Wd, bd):
    def tailk(x_ref, wd_ref, bd_ref, o_ref):
        x = x_ref[...]
        nrm = jnp.maximum(jnp.sqrt(jnp.sum(x * x)), 1e-12)
        o_ref[...] = (jnp.sum(x * wd_ref[...], axis=1, keepdims=True) / nrm
                      + bd_ref[...])

    return pl.pallas_call(
        tailk,
        out_shape=jax.ShapeDtypeStruct((1, 1), jnp.float32),
    )(pooled, Wd.reshape(1, -1), bd.reshape(1, 1))


def _expand_alpha(a):
    hh, f = a.shape
    d = hh * f
    cols = jnp.arange(16)[None, :]
    rowh = (jnp.arange(d) // f)[:, None]
    vals = a.reshape(d, 1)
    return jnp.where((cols == rowh) | (cols == rowh + 8), vals, 0.0)


def _pad_edges(ei):
    src = ei[0]
    dst = ei[1]
    pad = EPAD - E
    src = jnp.concatenate([src, jnp.zeros((pad,), jnp.int32)])
    dst = jnp.concatenate([dst, jnp.full((pad,), 65535, jnp.int32)])
    return src, dst


def kernel(node_feats, edge_index_int, edge_index_nh, W1, a1_src, a1_dst,
           W2, a2_src, a2_dst, Wd, bd):
    f32 = jnp.float32
    x1 = jnp.zeros((NP, 128), f32).at[:N, :11].set(node_feats)
    W1p = jnp.zeros((128, D1), f32).at[:11, :].set(W1)
    W2p = jnp.zeros((128, D2), f32).at[:D1, :].set(W2)
    ase1 = _expand_alpha(a1_src)
    ade1 = _expand_alpha(a1_dst)
    ase2 = _expand_alpha(a2_src)
    ade2 = _expand_alpha(a2_dst)

    si, di = _pad_edges(edge_index_int)
    sn, dn = _pad_edges(edge_index_nh)
    bs_i, bd_i, cnt_i = _bin_call(si, di)
    bs_n, bd_n, cnt_n = _bin_call(sn, dn)

    wh1, as1, ad1 = _prep_call(x1, W1p, ase1, ade1, D1)
    u1 = _u_call(as1, ad1).reshape(16)
    h1_i = _edge_call(D1, bs_i, bd_i, cnt_i, wh1, as1, ad1, u1)
    h1_n = _edge_call(D1, bs_n, bd_n, cnt_n, wh1, as1, ad1, u1)

    x2_i = jnp.pad(h1_i, ((0, 0), (0, 128 - D1)))
    wh2_i, as2_i, ad2_i = _prep_call(x2_i, W2p, ase2, ade2, D2)
    u2_i = _u_call(as2_i, ad2_i).reshape(16)
    h2_i = _edge_call(D2, bs_i, bd_i, cnt_i, wh2_i, as2_i, ad2_i, u2_i)

    x2_n = jnp.pad(h1_n, ((0, 0), (0, 128 - D1)))
    wh2_n, as2_n, ad2_n = _prep_call(x2_n, W2p, ase2, ade2, D2)
    u2_n = _u_call(as2_n, ad2_n).reshape(16)
    h2_n = _edge_call(D2, bs_n, bd_n, cnt_n, wh2_n, as2_n, ad2_n, u2_n)

    pooled = _pool_call(h2_i, h2_n)
    out = _tail_call(pooled, Wd, bd)
    return jnp.squeeze(out, 1)


# speedup vs baseline: 93.5385x; 1.1360x over previous
import functools

import jax
import jax.numpy as jnp
from jax import lax
from jax.experimental import pallas as pl
from jax.experimental.pallas import tpu as pltpu
from jax.experimental.pallas import tpu_sc as plsc

N = 50000
E = 800000
NP = 53248
BSZ = 4096
NBKT = 13
BSH = 12
NW = 32
EPW = 25600
EPAD = NW * EPW
RCAP = 28800
CH = 128
NHEAD = 6
D1, D2 = 96, 192
ROWBLK = 512


def _splat(s):
    return lax.broadcast_in_dim(s, (16,), ())


@functools.lru_cache(maxsize=None)
def _sc_mesh():
    return plsc.VectorSubcoreMesh(core_axis_name="c", subcore_axis_name="s")


_SC_PARAMS = pltpu.CompilerParams(needs_layout_passes=False,
                                  use_tc_tiling_on_sc=False)


def _bin_call(src_pad, dst_pad):
    @functools.partial(
        pl.kernel,
        out_type=(
            jax.ShapeDtypeStruct((NBKT + 1, NW * RCAP), jnp.int32),
            jax.ShapeDtypeStruct((NBKT + 1, NW * RCAP), jnp.int32),
            jax.ShapeDtypeStruct((NW, 16), jnp.int32),
        ),
        mesh=_sc_mesh(),
        compiler_params=_SC_PARAMS,
        scratch_types=dict(
            srcv=pltpu.VMEM((EPW,), jnp.int32),
            dstv=pltpu.VMEM((EPW,), jnp.int32),
            lsrc=pltpu.VMEM((RCAP,), jnp.int32),
            ldst=pltpu.VMEM((RCAP,), jnp.int32),
            cntv=pltpu.VMEM((16,), jnp.int32),
        ),
    )
    def bink(src_h, dst_h, bsrc_h, bdst_h, cnt_h, srcv, dstv, lsrc, ldst,
             cntv):
        cid = lax.axis_index("c")
        sid = lax.axis_index("s")
        w = sid * 2 + cid
        lane = lax.iota(jnp.int32, 16)
        base = w * EPW
        for c in range(8):
            pltpu.sync_copy(src_h.at[pl.ds(base + c * 3200, 3200)],
                            srcv.at[pl.ds(c * 3200, 3200)])
            pltpu.sync_copy(dst_h.at[pl.ds(base + c * 3200, 3200)],
                            dstv.at[pl.ds(c * 3200, 3200)])
        cvec = jnp.zeros((16,), jnp.int32)
        for b in range(NBKT):
            def grp(g, cur, b=b):
                dv = dstv[pl.ds(g * 16, 16)]
                sv = srcv[pl.ds(g * 16, 16)]
                mask = (dv >> BSH) == b
                sk, svv, _ = plsc.sort_key_val(dv, sv, mask=mask)
                ldst[pl.ds(cur, 16)] = sk
                lsrc[pl.ds(cur, 16)] = svv
                pc = plsc.all_reduce_population_count(mask)
                return cur + pc[0]

            cursor = lax.fori_loop(0, EPW // 16, grp, jnp.int32(0))
            pdst = _splat(jnp.int32(b * BSZ)) + lane
            psrc = lane
            for pg in range(8):
                ldst[pl.ds(cursor + pg * 16, 16)] = pdst
                lsrc[pl.ds(cursor + pg * 16, 16)] = psrc
            rounded = ((cursor + 127) >> 7) << 7
            trips = (rounded + 2047) >> 11
            rbase = w * RCAP

            def dout(c, _, b=b):
                pltpu.sync_copy(lsrc.at[pl.ds(c * 2048, 2048)],
                                bsrc_h.at[b, pl.ds(rbase + c * 2048, 2048)])
                pltpu.sync_copy(ldst.at[pl.ds(c * 2048, 2048)],
                                bdst_h.at[b, pl.ds(rbase + c * 2048, 2048)])
                return 0

            lax.fori_loop(0, trips, dout, 0)
            cvec = jnp.where(lane == b, _splat(cursor), cvec)
        cntv[...] = cvec
        pltpu.sync_copy(cntv, cnt_h.at[w])

    return bink(src_pad, dst_pad)


def _edge_call(D, bsrc, bdst, cnt, wh, a_s, a_d, u16):
    nvreg = D // 16
    fdim = D // NHEAD
    head_of = [(k * 16) // fdim for k in range(nvreg)]

    @functools.partial(
        pl.kernel,
        out_type=jax.ShapeDtypeStruct((NP, D), jnp.float32),
        mesh=_sc_mesh(),
        compiler_params=_SC_PARAMS,
        scratch_types=dict(
            cntv=pltpu.VMEM((NW, 16), jnp.int32),
            uv=pltpu.VMEM((16,), jnp.float32),
            srcb=pltpu.VMEM((CH,), jnp.int32),
            dstb=pltpu.VMEM((CH,), jnp.int32),
            locb=pltpu.VMEM((CH,), jnp.int32),
            asb=pltpu.VMEM((CH, 16), jnp.float32),
            adb=pltpu.VMEM((CH, 16), jnp.float32),
            mb=pltpu.VMEM((CH, 16), jnp.float32),
            rows=pltpu.VMEM((CH, D), jnp.float32),
            zbuf=pltpu.VMEM((64, D), jnp.float32),
            zden=pltpu.VMEM((64, 16), jnp.float32),
            fb=pltpu.VMEM((64, D), jnp.float32),
            dnb=pltpu.VMEM((64, 16), jnp.float32),
            acc_sh=pltpu.VMEM_SHARED((BSZ, D), jnp.float32),
            den_sh=pltpu.VMEM_SHARED((BSZ, 16), jnp.float32),
            sem1=pltpu.SemaphoreType.DMA,
            sem2=pltpu.SemaphoreType.DMA,
            sem3=pltpu.SemaphoreType.DMA,
        ),
    )
    def edgek(bsrc_h, bdst_h, cnt_h, wh_h, as_h, ad_h, u_h, hout_h,
              cntv, uv, srcb, dstb, locb, asb, adb, mb, rows, zbuf, zden,
              fb, dnb, acc_sh, den_sh, sem1, sem2, sem3):
        cid = lax.axis_index("c")
        sid = lax.axis_index("s")
        lane = lax.iota(jnp.int32, 16)
        zero16 = jnp.zeros((16,), jnp.float32)
        hidx = [jnp.full((16,), h, jnp.int32) for h in range(NHEAD)]
        pltpu.sync_copy(cnt_h, cntv)
        pltpu.sync_copy(u_h, uv)
        uvv = uv[pl.ds(0, 16)]

        def zrow(r, _):
            for k in range(nvreg):
                zbuf[r, pl.ds(k * 16, 16)] = zero16
            zden[r, pl.ds(0, 16)] = zero16
            return 0

        lax.fori_loop(0, 64, zrow, 0)

        def slot_body(slot, _):
            b = slot * 2 + cid

            @pl.when(b < NBKT)
            def _process():
                def zc(i, _):
                    pltpu.sync_copy(zbuf, acc_sh.at[pl.ds(sid * 256 + i * 64,
                                                          64)])
                    pltpu.sync_copy(zden, den_sh.at[pl.ds(sid * 256 + i * 64,
                                                          64)])
                    return 0

                lax.fori_loop(0, 4, zc, 0)
                plsc.subcore_barrier()

                for wo in range(2):
                    w = sid * 2 + wo
                    crow = cntv[w, pl.ds(0, 16)]
                    cnt_wb = jnp.take(crow, _splat(b))[0]
                    trips = (cnt_wb + (CH - 1)) >> 7

                    def chunk(c, _, b=b, w=w, cnt_wb=cnt_wb):
                        off = w * RCAP + c * CH
                        pltpu.sync_copy(bsrc_h.at[b, pl.ds(off, CH)], srcb)
                        pltpu.sync_copy(bdst_h.at[b, pl.ds(off, CH)], dstb)
                        cp1 = pltpu.async_copy(as_h.at[srcb], asb, sem1)
                        cp2 = pltpu.async_copy(ad_h.at[dstb], adb, sem2)
                        cp3 = pltpu.async_copy(wh_h.at[srcb], rows, sem3)

                        @plsc.parallel_loop(0, CH // 16, unroll=2)
                        def lg(g):
                            dv = dstb[pl.ds(g * 16, 16)]
                            locb[pl.ds(g * 16, 16)] = dv & (BSZ - 1)
                        cp1.wait()
                        cp2.wait()
                        ebase = c * CH

                        @plsc.parallel_loop(0, CH, unroll=4)
                        def edge_m(e):
                            asv = asb[e, pl.ds(0, 16)]
                            adv = adb[e, pl.ds(0, 16)]
                            s = asv + adv
                            ev = jnp.maximum(s, 0.2 * s) - uvv
                            m = jnp.exp(ev)
                            valid = _splat(ebase + e) < _splat(cnt_wb)
                            m = jnp.where(valid, m, 0.0)
                            mb[e, pl.ds(0, 16)] = m
                        cp3.wait()

                        @plsc.parallel_loop(0, CH, unroll=2)
                        def edge_scale(e):
                            m = mb[e, pl.ds(0, 16)]
                            sps = [jnp.take(m, hidx[h]) for h in range(NHEAD)]
                            for k in range(nvreg):
                                r = rows[e, pl.ds(k * 16, 16)]
                                rows[e, pl.ds(k * 16, 16)] = r * sps[head_of[k]]
                        pltpu.sync_copy(mb, den_sh.at[locb], add=True)
                        pltpu.sync_copy(rows, acc_sh.at[locb], add=True)
                        return 0

                    lax.fori_loop(0, trips, chunk, 0)
                plsc.subcore_barrier()

                def fin(i, _):
                    r0 = sid * 256 + i * 64
                    pltpu.sync_copy(acc_sh.at[pl.ds(r0, 64)], fb)
                    pltpu.sync_copy(den_sh.at[pl.ds(r0, 64)], dnb)

                    @plsc.parallel_loop(0, 64, unroll=2)
                    def frow(n):
                        dv = dnb[n, pl.ds(0, 16)]
                        rec = 1.0 / (dv + 1e-16)
                        sps = [jnp.take(rec, hidx[h]) for h in range(NHEAD)]
                        for k in range(nvreg):
                            x = fb[n, pl.ds(k * 16, 16)] * sps[head_of[k]]
                            y = jnp.where(x > 0, x, jnp.exp(x) - 1.0)
                            fb[n, pl.ds(k * 16, 16)] = y
                    pltpu.sync_copy(fb, hout_h.at[pl.ds(b * BSZ + r0, 64)])
                    return 0

                lax.fori_loop(0, 4, fin, 0)
                plsc.subcore_barrier()

            return 0

        lax.fori_loop(0, (NBKT + 1) // 2, slot_body, 0)

    return edgek(bsrc, bdst, cnt, wh, a_s, a_d, u16)


def _prep_call(x_pad, w_pad, ase, ade, D):
    kdim = x_pad.shape[1]
    grid = NP // ROWBLK

    def prep(x_ref, w_ref, ase_ref, ade_ref, wh_ref, as_ref, ad_ref):
        xb = x_ref[...]
        whb = jnp.dot(xb, w_ref[...], preferred_element_type=jnp.float32)
        wh_ref[...] = whb
        as_ref[...] = jnp.dot(whb, ase_ref[...],
                              preferred_element_type=jnp.float32)
        ad_ref[...] = jnp.dot(whb, ade_ref[...],
                              preferred_element_type=jnp.float32)

    return pl.pallas_call(
        prep,
        grid=(grid,),
        in_specs=[
            pl.BlockSpec((ROWBLK, kdim), lambda i: (i, 0)),
            pl.BlockSpec((kdim, D), lambda i: (0, 0)),
            pl.BlockSpec((D, 16), lambda i: (0, 0)),
            pl.BlockSpec((D, 16), lambda i: (0, 0)),
        ],
        out_specs=[
            pl.BlockSpec((ROWBLK, D), lambda i: (i, 0)),
            pl.BlockSpec((ROWBLK, 16), lambda i: (i, 0)),
            pl.BlockSpec((ROWBLK, 16), lambda i: (i, 0)),
        ],
        out_shape=[
            jax.ShapeDtypeStruct((NP, D), jnp.float32),
            jax.ShapeDtypeStruct((NP, 16), jnp.float32),
            jax.ShapeDtypeStruct((NP, 16), jnp.float32),
        ],
    )(x_pad, w_pad, ase, ade)


def _u_call(a_s, a_d):
    def uk(as_ref, ad_ref, u_ref):
        u = (jnp.max(as_ref[...], axis=0, keepdims=True)
             + jnp.max(ad_ref[...], axis=0, keepdims=True))
        u_ref[...] = jnp.maximum(u, 0.2 * u)

    return pl.pallas_call(
        uk,
        out_shape=jax.ShapeDtypeStruct((1, 16), jnp.float32),
    )(a_s, a_d)


def _pool_call(h_i, h_n):
    grid = NP // ROWBLK

    def poolk(hi_ref, hn_ref, o_ref):
        @pl.when(pl.program_id(0) == 0)
        def _init():
            o_ref[...] = jnp.zeros_like(o_ref)

        s1 = jnp.sum(hi_ref[...], axis=0, keepdims=True)
        s2 = jnp.sum(hn_ref[...], axis=0, keepdims=True)
        o_ref[...] += jnp.concatenate([s1, s2], axis=1)

    return pl.pallas_call(
        poolk,
        grid=(grid,),
        in_specs=[
            pl.BlockSpec((ROWBLK, D2), lambda i: (i, 0)),
            pl.BlockSpec((ROWBLK, D2), lambda i: (i, 0)),
        ],
        out_specs=pl.BlockSpec((1, 2 * D2), lambda i: (0, 0)),
        out_shape=jax.ShapeDtypeStruct((1, 2 * D2), jnp.float32),
    )(h_i, h_n)


def _tail_call(pooled, Wd, bd):
    def tailk(x_ref, wd_ref, bd_ref, o_ref):
        x = x_ref[...]
        nrm = jnp.maximum(jnp.sqrt(jnp.sum(x * x)), 1e-12)
        o_ref[...] = (jnp.sum(x * wd_ref[...], axis=1, keepdims=True) / nrm
                      + bd_ref[...])

    return pl.pallas_call(
        tailk,
        out_shape=jax.ShapeDtypeStruct((1, 1), jnp.float32),
    )(pooled, Wd.reshape(1, -1), bd.reshape(1, 1))


def _expand_alpha(a):
    hh, f = a.shape
    d = hh * f
    cols = jnp.arange(16)[None, :]
    rowh = (jnp.arange(d) // f)[:, None]
    vals = a.reshape(d, 1)
    return jnp.where((cols == rowh) | (cols == rowh + 8), vals, 0.0)


def _pad_edges(ei):
    src = ei[0]
    dst = ei[1]
    pad = EPAD - E
    src = jnp.concatenate([src, jnp.zeros((pad,), jnp.int32)])
    dst = jnp.concatenate([dst, jnp.full((pad,), 65535, jnp.int32)])
    return src, dst


def kernel(node_feats, edge_index_int, edge_index_nh, W1, a1_src, a1_dst,
           W2, a2_src, a2_dst, Wd, bd):
    f32 = jnp.float32
    x1 = jnp.zeros((NP, 128), f32).at[:N, :11].set(node_feats)
    W1p = jnp.zeros((128, D1), f32).at[:11, :].set(W1)
    W2p = jnp.zeros((128, D2), f32).at[:D1, :].set(W2)
    ase1 = _expand_alpha(a1_src)
    ade1 = _expand_alpha(a1_dst)
    ase2 = _expand_alpha(a2_src)
    ade2 = _expand_alpha(a2_dst)

    si, di = _pad_edges(edge_index_int)
    sn, dn = _pad_edges(edge_index_nh)
    bs_i, bd_i, cnt_i = _bin_call(si, di)
    bs_n, bd_n, cnt_n = _bin_call(sn, dn)

    wh1, as1, ad1 = _prep_call(x1, W1p, ase1, ade1, D1)
    u1 = _u_call(as1, ad1).reshape(16)
    h1_i = _edge_call(D1, bs_i, bd_i, cnt_i, wh1, as1, ad1, u1)
    h1_n = _edge_call(D1, bs_n, bd_n, cnt_n, wh1, as1, ad1, u1)

    x2_i = jnp.pad(h1_i, ((0, 0), (0, 128 - D1)))
    wh2_i, as2_i, ad2_i = _prep_call(x2_i, W2p, ase2, ade2, D2)
    u2_i = _u_call(as2_i, ad2_i).reshape(16)
    h2_i = _edge_call(D2, bs_i, bd_i, cnt_i, wh2_i, as2_i, ad2_i, u2_i)

    x2_n = jnp.pad(h1_n, ((0, 0), (0, 128 - D1)))
    wh2_n, as2_n, ad2_n = _prep_call(x2_n, W2p, ase2, ade2, D2)
    u2_n = _u_call(as2_n, ad2_n).reshape(16)
    h2_n = _edge_call(D2, bs_n, bd_n, cnt_n, wh2_n, as2_n, ad2_n, u2_n)

    pooled = _pool_call(h2_i, h2_n)
    out = _tail_call(pooled, Wd, bd)
    return jnp.squeeze(out, 1)


# speedup vs baseline: 103.1213x; 1.1024x over previous
import functools

import jax
import jax.numpy as jnp
from jax import lax
from jax.experimental import pallas as pl
from jax.experimental.pallas import tpu as pltpu
from jax.experimental.pallas import tpu_sc as plsc

N = 50000
E = 800000
NP = 53248
BSZ = 4096
NBKT = 13
BSH = 12
NW = 32
EPW = 25600
EPAD = NW * EPW
RCAP = 28800
CH = 128
NHEAD = 6
D1, D2 = 96, 192
ROWBLK = 512


def _splat(s):
    return lax.broadcast_in_dim(s, (16,), ())


@functools.lru_cache(maxsize=None)
def _sc_mesh():
    return plsc.VectorSubcoreMesh(core_axis_name="c", subcore_axis_name="s")


_SC_PARAMS = pltpu.CompilerParams(needs_layout_passes=False,
                                  use_tc_tiling_on_sc=False)


def _bin_call(src_pad, dst_pad):
    @functools.partial(
        pl.kernel,
        out_type=(
            jax.ShapeDtypeStruct((NBKT + 1, NW * RCAP), jnp.int32),
            jax.ShapeDtypeStruct((NBKT + 1, NW * RCAP), jnp.int32),
            jax.ShapeDtypeStruct((NW, 16), jnp.int32),
        ),
        mesh=_sc_mesh(),
        compiler_params=_SC_PARAMS,
        scratch_types=dict(
            srcv=pltpu.VMEM((EPW,), jnp.int32),
            dstv=pltpu.VMEM((EPW,), jnp.int32),
            lsrc=pltpu.VMEM((RCAP,), jnp.int32),
            ldst=pltpu.VMEM((RCAP,), jnp.int32),
            cntv=pltpu.VMEM((16,), jnp.int32),
        ),
    )
    def bink(src_h, dst_h, bsrc_h, bdst_h, cnt_h, srcv, dstv, lsrc, ldst,
             cntv):
        cid = lax.axis_index("c")
        sid = lax.axis_index("s")
        w = sid * 2 + cid
        lane = lax.iota(jnp.int32, 16)
        base = w * EPW
        for c in range(8):
            pltpu.sync_copy(src_h.at[pl.ds(base + c * 3200, 3200)],
                            srcv.at[pl.ds(c * 3200, 3200)])
            pltpu.sync_copy(dst_h.at[pl.ds(base + c * 3200, 3200)],
                            dstv.at[pl.ds(c * 3200, 3200)])
        cvec = jnp.zeros((16,), jnp.int32)
        for b in range(NBKT):
            def grp(g, cur, b=b):
                dv = dstv[pl.ds(g * 16, 16)]
                sv = srcv[pl.ds(g * 16, 16)]
                mask = (dv >> BSH) == b
                sk, svv, _ = plsc.sort_key_val(dv, sv, mask=mask)
                ldst[pl.ds(cur, 16)] = sk
                lsrc[pl.ds(cur, 16)] = svv
                pc = plsc.all_reduce_population_count(mask)
                return cur + pc[0]

            cursor = lax.fori_loop(0, EPW // 16, grp, jnp.int32(0))
            pdst = _splat(jnp.int32(b * BSZ)) + lane
            psrc = lane
            for pg in range(8):
                ldst[pl.ds(cursor + pg * 16, 16)] = pdst
                lsrc[pl.ds(cursor + pg * 16, 16)] = psrc
            rounded = ((cursor + 127) >> 7) << 7
            trips = (rounded + 2047) >> 11
            rbase = w * RCAP

            def dout(c, _, b=b):
                pltpu.sync_copy(lsrc.at[pl.ds(c * 2048, 2048)],
                                bsrc_h.at[b, pl.ds(rbase + c * 2048, 2048)])
                pltpu.sync_copy(ldst.at[pl.ds(c * 2048, 2048)],
                                bdst_h.at[b, pl.ds(rbase + c * 2048, 2048)])
                return 0

            lax.fori_loop(0, trips, dout, 0)
            cvec = jnp.where(lane == b, _splat(cursor), cvec)
        cntv[...] = cvec
        pltpu.sync_copy(cntv, cnt_h.at[w])

    return bink(src_pad, dst_pad)


def _edge_call(D, bsrc, bdst, cnt, wh, a_s, a_d, u16):
    nvreg = D // 16
    fdim = D // NHEAD
    head_of = [(k * 16) // fdim for k in range(nvreg)]
    CE = 64

    scr = dict(
        cntv=pltpu.VMEM((NW, 16), jnp.int32),
        uv=pltpu.VMEM((16,), jnp.float32),
        zbuf=pltpu.VMEM((64, D), jnp.float32),
        zden=pltpu.VMEM((64, 16), jnp.float32),
        fb=pltpu.VMEM((64, D), jnp.float32),
        dnb=pltpu.VMEM((64, 16), jnp.float32),
        acc_sh=pltpu.VMEM_SHARED((BSZ, D), jnp.float32),
        den_sh=pltpu.VMEM_SHARED((BSZ, 16), jnp.float32),
    )
    for p in (0, 1):
        scr.update({
            f"srcb{p}": pltpu.VMEM((CE,), jnp.int32),
            f"dstb{p}": pltpu.VMEM((CE,), jnp.int32),
            f"locb{p}": pltpu.VMEM((CE,), jnp.int32),
            f"asb{p}": pltpu.VMEM((CE, 16), jnp.float32),
            f"adb{p}": pltpu.VMEM((CE, 16), jnp.float32),
            f"mb{p}": pltpu.VMEM((CE, 16), jnp.float32),
            f"rows{p}": pltpu.VMEM((CE, D), jnp.float32),
            f"sga{p}": pltpu.SemaphoreType.DMA,
            f"sgd{p}": pltpu.SemaphoreType.DMA,
            f"sgw{p}": pltpu.SemaphoreType.DMA,
            f"ssm{p}": pltpu.SemaphoreType.DMA,
            f"ssr{p}": pltpu.SemaphoreType.DMA,
        })

    @functools.partial(
        pl.kernel,
        out_type=jax.ShapeDtypeStruct((NP, D), jnp.float32),
        mesh=_sc_mesh(),
        compiler_params=_SC_PARAMS,
        scratch_types=scr,
    )
    def edgek(bsrc_h, bdst_h, cnt_h, wh_h, as_h, ad_h, u_h, hout_h, **s):
        cid = lax.axis_index("c")
        sid = lax.axis_index("s")
        lane = lax.iota(jnp.int32, 16)
        zero16 = jnp.zeros((16,), jnp.float32)
        hidx = [jnp.full((16,), h, jnp.int32) for h in range(NHEAD)]
        cntv, uv = s["cntv"], s["uv"]
        zbuf, zden, fb, dnb = s["zbuf"], s["zden"], s["fb"], s["dnb"]
        acc_sh, den_sh = s["acc_sh"], s["den_sh"]
        bufs = [
            tuple(s[f"{n}{p}"] for n in
                  ("srcb", "dstb", "locb", "asb", "adb", "mb", "rows",
                   "sga", "sgd", "sgw", "ssm", "ssr"))
            for p in (0, 1)
        ]
        pltpu.sync_copy(cnt_h, cntv)
        pltpu.sync_copy(u_h, uv)
        uvv = uv[pl.ds(0, 16)]

        def zrow(r, _):
            for k in range(nvreg):
                zbuf[r, pl.ds(k * 16, 16)] = zero16
            zden[r, pl.ds(0, 16)] = zero16
            return 0

        lax.fori_loop(0, 64, zrow, 0)

        def slot_body(slot, _):
            b = slot * 2 + cid

            @pl.when(b < NBKT)
            def _process():
                def zc(i, _):
                    pltpu.sync_copy(zbuf, acc_sh.at[pl.ds(sid * 256 + i * 64,
                                                          64)])
                    pltpu.sync_copy(zden, den_sh.at[pl.ds(sid * 256 + i * 64,
                                                          64)])
                    return 0

                lax.fori_loop(0, 4, zc, 0)
                plsc.subcore_barrier()

                for wo in range(2):
                    w = sid * 2 + wo
                    crow = cntv[w, pl.ds(0, 16)]
                    cnt_wb = jnp.take(crow, _splat(b))[0]
                    trips = (cnt_wb + (CE - 1)) >> 6
                    pairs = (trips + 1) >> 1

                    def pair(t, _, b=b, w=w, cnt_wb=cnt_wb, trips=trips):
                        gath = [None, None]
                        scat = [None, None]
                        for half in (0, 1):
                            c = 2 * t + half
                            (srcb, dstb, locb, asb, adb, mb, rows,
                             sga, sgd, sgw, ssm, ssr) = bufs[half]

                            @pl.when(c < trips)
                            def _issue(c=c, srcb=srcb, dstb=dstb, half=half):
                                off = w * RCAP + c * CE
                                pltpu.sync_copy(bsrc_h.at[b, pl.ds(off, CE)],
                                                srcb)
                                pltpu.sync_copy(bdst_h.at[b, pl.ds(off, CE)],
                                                dstb)

                            @pl.when(c < trips)
                            def _gath(c=c, half=half):
                                cp1 = pltpu.async_copy(as_h.at[srcb], asb, sga)
                                cp2 = pltpu.async_copy(ad_h.at[dstb], adb, sgd)
                                cp3 = pltpu.async_copy(wh_h.at[srcb], rows,
                                                       sgw)

                        for half in (0, 1):
                            c = 2 * t + half
                            (srcb, dstb, locb, asb, adb, mb, rows,
                             sga, sgd, sgw, ssm, ssr) = bufs[half]

                            @pl.when(c < trips)
                            def _compute(c=c, srcb=srcb, dstb=dstb, locb=locb,
                                         asb=asb, adb=adb, mb=mb, rows=rows,
                                         sga=sga, sgd=sgd, sgw=sgw, ssm=ssm,
                                         ssr=ssr):
                                @plsc.parallel_loop(0, CE // 16, unroll=2)
                                def lg(g):
                                    dv = dstb[pl.ds(g * 16, 16)]
                                    locb[pl.ds(g * 16, 16)] = dv & (BSZ - 1)

                                pltpu.make_async_copy(as_h.at[srcb], asb,
                                                      sga).wait()
                                pltpu.make_async_copy(ad_h.at[dstb], adb,
                                                      sgd).wait()
                                ebase = c * CE

                                @plsc.parallel_loop(0, CE, unroll=4)
                                def edge_m(e):
                                    asv = asb[e, pl.ds(0, 16)]
                                    adv = adb[e, pl.ds(0, 16)]
                                    sv = asv + adv
                                    ev = jnp.maximum(sv, 0.2 * sv) - uvv
                                    m = jnp.exp(ev)
                                    valid = _splat(ebase + e) < _splat(cnt_wb)
                                    m = jnp.where(valid, m, 0.0)
                                    mb[e, pl.ds(0, 16)] = m

                                pltpu.make_async_copy(wh_h.at[srcb], rows,
                                                      sgw).wait()

                                @plsc.parallel_loop(0, CE, unroll=2)
                                def edge_scale(e):
                                    m = mb[e, pl.ds(0, 16)]
                                    sps = [jnp.take(m, hidx[h])
                                           for h in range(NHEAD)]
                                    for k in range(nvreg):
                                        r = rows[e, pl.ds(k * 16, 16)]
                                        rows[e, pl.ds(k * 16, 16)] = (
                                            r * sps[head_of[k]])

                                pltpu.async_copy(mb, den_sh.at[locb], ssm,
                                                 add=True)
                                pltpu.async_copy(rows, acc_sh.at[locb], ssr,
                                                 add=True)

                        for half in (0, 1):
                            c = 2 * t + half
                            (srcb, dstb, locb, asb, adb, mb, rows,
                             sga, sgd, sgw, ssm, ssr) = bufs[half]

                            @pl.when(c < trips)
                            def _drain(mb=mb, rows=rows, locb=locb, ssm=ssm,
                                       ssr=ssr):
                                pltpu.make_async_copy(
                                    mb, den_sh.at[locb], ssm).wait()
                                pltpu.make_async_copy(
                                    rows, acc_sh.at[locb], ssr).wait()
                        return 0

                    lax.fori_loop(0, pairs, pair, 0)
                plsc.subcore_barrier()

                def fin(i, _):
                    r0 = sid * 256 + i * 64
                    pltpu.sync_copy(acc_sh.at[pl.ds(r0, 64)], fb)
                    pltpu.sync_copy(den_sh.at[pl.ds(r0, 64)], dnb)

                    @plsc.parallel_loop(0, 64, unroll=2)
                    def frow(n):
                        dv = dnb[n, pl.ds(0, 16)]
                        rec = 1.0 / (dv + 1e-16)
                        sps = [jnp.take(rec, hidx[h]) for h in range(NHEAD)]
                        for k in range(nvreg):
                            x = fb[n, pl.ds(k * 16, 16)] * sps[head_of[k]]
                            y = jnp.where(x > 0, x, jnp.exp(x) - 1.0)
                            fb[n, pl.ds(k * 16, 16)] = y
                    pltpu.sync_copy(fb, hout_h.at[pl.ds(b * BSZ + r0, 64)])
                    return 0

                lax.fori_loop(0, 4, fin, 0)
                plsc.subcore_barrier()

            return 0

        lax.fori_loop(0, (NBKT + 1) // 2, slot_body, 0)

    return edgek(bsrc, bdst, cnt, wh, a_s, a_d, u16)


def _prep_call(x_pad, w_pad, ase, ade, D):
    kdim = x_pad.shape[1]
    grid = NP // ROWBLK

    def prep(x_ref, w_ref, ase_ref, ade_ref, wh_ref, as_ref, ad_ref):
        xb = x_ref[...]
        whb = jnp.dot(xb, w_ref[...], preferred_element_type=jnp.float32)
        wh_ref[...] = whb
        as_ref[...] = jnp.dot(whb, ase_ref[...],
                              preferred_element_type=jnp.float32)
        ad_ref[...] = jnp.dot(whb, ade_ref[...],
                              preferred_element_type=jnp.float32)

    return pl.pallas_call(
        prep,
        grid=(grid,),
        in_specs=[
            pl.BlockSpec((ROWBLK, kdim), lambda i: (i, 0)),
            pl.BlockSpec((kdim, D), lambda i: (0, 0)),
            pl.BlockSpec((D, 16), lambda i: (0, 0)),
            pl.BlockSpec((D, 16), lambda i: (0, 0)),
        ],
        out_specs=[
            pl.BlockSpec((ROWBLK, D), lambda i: (i, 0)),
            pl.BlockSpec((ROWBLK, 16), lambda i: (i, 0)),
            pl.BlockSpec((ROWBLK, 16), lambda i: (i, 0)),
        ],
        out_shape=[
            jax.ShapeDtypeStruct((NP, D), jnp.float32),
            jax.ShapeDtypeStruct((NP, 16), jnp.float32),
            jax.ShapeDtypeStruct((NP, 16), jnp.float32),
        ],
    )(x_pad, w_pad, ase, ade)


def _u_call(a_s, a_d):
    def uk(as_ref, ad_ref, u_ref):
        u = (jnp.max(as_ref[...], axis=0, keepdims=True)
             + jnp.max(ad_ref[...], axis=0, keepdims=True))
        u_ref[...] = jnp.maximum(u, 0.2 * u)

    return pl.pallas_call(
        uk,
        out_shape=jax.ShapeDtypeStruct((1, 16), jnp.float32),
    )(a_s, a_d)


def _pool_call(h_i, h_n):
    grid = NP // ROWBLK

    def poolk(hi_ref, hn_ref, o_ref):
        @pl.when(pl.program_id(0) == 0)
        def _init():
            o_ref[...] = jnp.zeros_like(o_ref)

        s1 = jnp.sum(hi_ref[...], axis=0, keepdims=True)
        s2 = jnp.sum(hn_ref[...], axis=0, keepdims=True)
        o_ref[...] += jnp.concatenate([s1, s2], axis=1)

    return pl.pallas_call(
        poolk,
        grid=(grid,),
        in_specs=[
            pl.BlockSpec((ROWBLK, D2), lambda i: (i, 0)),
            pl.BlockSpec((ROWBLK, D2), lambda i: (i, 0)),
        ],
        out_specs=pl.BlockSpec((1, 2 * D2), lambda i: (0, 0)),
        out_shape=jax.ShapeDtypeStruct((1, 2 * D2), jnp.float32),
    )(h_i, h_n)


def _tail_call(pooled, Wd, bd):
    def tailk(x_ref, wd_ref, bd_ref, o_ref):
        x = x_ref[...]
        nrm = jnp.maximum(jnp.sqrt(jnp.sum(x * x)), 1e-12)
        o_ref[...] = (jnp.sum(x * wd_ref[...], axis=1, keepdims=True) / nrm
                      + bd_ref[...])

    return pl.pallas_call(
        tailk,
        out_shape=jax.ShapeDtypeStruct((1, 1), jnp.float32),
    )(pooled, Wd.reshape(1, -1), bd.reshape(1, 1))


def _expand_alpha(a):
    hh, f = a.shape
    d = hh * f
    cols = jnp.arange(16)[None, :]
    rowh = (jnp.arange(d) // f)[:, None]
    vals = a.reshape(d, 1)
    return jnp.where((cols == rowh) | (cols == rowh + 8), vals, 0.0)


def _pad_edges(ei):
    src = ei[0]
    dst = ei[1]
    pad = EPAD - E
    src = jnp.concatenate([src, jnp.zeros((pad,), jnp.int32)])
    dst = jnp.concatenate([dst, jnp.full((pad,), 65535, jnp.int32)])
    return src, dst


def kernel(node_feats, edge_index_int, edge_index_nh, W1, a1_src, a1_dst,
           W2, a2_src, a2_dst, Wd, bd):
    f32 = jnp.float32
    x1 = jnp.zeros((NP, 128), f32).at[:N, :11].set(node_feats)
    W1p = jnp.zeros((128, D1), f32).at[:11, :].set(W1)
    W2p = jnp.zeros((128, D2), f32).at[:D1, :].set(W2)
    ase1 = _expand_alpha(a1_src)
    ade1 = _expand_alpha(a1_dst)
    ase2 = _expand_alpha(a2_src)
    ade2 = _expand_alpha(a2_dst)

    si, di = _pad_edges(edge_index_int)
    sn, dn = _pad_edges(edge_index_nh)
    bs_i, bd_i, cnt_i = _bin_call(si, di)
    bs_n, bd_n, cnt_n = _bin_call(sn, dn)

    wh1, as1, ad1 = _prep_call(x1, W1p, ase1, ade1, D1)
    u1 = _u_call(as1, ad1).reshape(16)
    h1_i = _edge_call(D1, bs_i, bd_i, cnt_i, wh1, as1, ad1, u1)
    h1_n = _edge_call(D1, bs_n, bd_n, cnt_n, wh1, as1, ad1, u1)

    x2_i = jnp.pad(h1_i, ((0, 0), (0, 128 - D1)))
    wh2_i, as2_i, ad2_i = _prep_call(x2_i, W2p, ase2, ade2, D2)
    u2_i = _u_call(as2_i, ad2_i).reshape(16)
    h2_i = _edge_call(D2, bs_i, bd_i, cnt_i, wh2_i, as2_i, ad2_i, u2_i)

    x2_n = jnp.pad(h1_n, ((0, 0), (0, 128 - D1)))
    wh2_n, as2_n, ad2_n = _prep_call(x2_n, W2p, ase2, ade2, D2)
    u2_n = _u_call(as2_n, ad2_n).reshape(16)
    h2_n = _edge_call(D2, bs_n, bd_n, cnt_n, wh2_n, as2_n, ad2_n, u2_n)

    pooled = _pool_call(h2_i, h2_n)
    out = _tail_call(pooled, Wd, bd)
    return jnp.squeeze(out, 1)


# speedup vs baseline: 103.3395x; 1.0021x over previous
import functools

import jax
import jax.numpy as jnp
from jax import lax
from jax.experimental import pallas as pl
from jax.experimental.pallas import tpu as pltpu
from jax.experimental.pallas import tpu_sc as plsc

N = 50000
E = 800000
NP = 53248
BSZ = 4096
NBKT = 13
BSH = 12
NW = 32
EPW = 25600
EPAD = NW * EPW
RCAP = 28800
CH = 128
NHEAD = 6
D1, D2 = 96, 192
ROWBLK = 512


def _splat(s):
    return lax.broadcast_in_dim(s, (16,), ())


@functools.lru_cache(maxsize=None)
def _sc_mesh():
    return plsc.VectorSubcoreMesh(core_axis_name="c", subcore_axis_name="s")


_SC_PARAMS = pltpu.CompilerParams(needs_layout_passes=False,
                                  use_tc_tiling_on_sc=False)


def _bin_call(src_pad, dst_pad):
    @functools.partial(
        pl.kernel,
        out_type=(
            jax.ShapeDtypeStruct((NBKT + 1, NW * RCAP), jnp.int32),
            jax.ShapeDtypeStruct((NBKT + 1, NW * RCAP), jnp.int32),
            jax.ShapeDtypeStruct((NW, 16), jnp.int32),
        ),
        mesh=_sc_mesh(),
        compiler_params=_SC_PARAMS,
        scratch_types=dict(
            srcv=pltpu.VMEM((EPW,), jnp.int32),
            dstv=pltpu.VMEM((EPW,), jnp.int32),
            lsrc=pltpu.VMEM((RCAP,), jnp.int32),
            ldst=pltpu.VMEM((RCAP,), jnp.int32),
            cntv=pltpu.VMEM((16,), jnp.int32),
        ),
    )
    def bink(src_h, dst_h, bsrc_h, bdst_h, cnt_h, srcv, dstv, lsrc, ldst,
             cntv):
        cid = lax.axis_index("c")
        sid = lax.axis_index("s")
        w = sid * 2 + cid
        lane = lax.iota(jnp.int32, 16)
        base = w * EPW
        for c in range(8):
            pltpu.sync_copy(src_h.at[pl.ds(base + c * 3200, 3200)],
                            srcv.at[pl.ds(c * 3200, 3200)])
            pltpu.sync_copy(dst_h.at[pl.ds(base + c * 3200, 3200)],
                            dstv.at[pl.ds(c * 3200, 3200)])
        cvec = jnp.zeros((16,), jnp.int32)
        for b in range(NBKT):
            def grp(g, cur, b=b):
                dv = dstv[pl.ds(g * 16, 16)]
                sv = srcv[pl.ds(g * 16, 16)]
                mask = (dv >> BSH) == b
                sk, svv, _ = plsc.sort_key_val(dv, sv, mask=mask)
                ldst[pl.ds(cur, 16)] = sk
                lsrc[pl.ds(cur, 16)] = svv
                pc = plsc.all_reduce_population_count(mask)
                return cur + pc[0]

            cursor = lax.fori_loop(0, EPW // 16, grp, jnp.int32(0))
            pdst = _splat(jnp.int32(b * BSZ)) + lane
            psrc = lane
            for pg in range(8):
                ldst[pl.ds(cursor + pg * 16, 16)] = pdst
                lsrc[pl.ds(cursor + pg * 16, 16)] = psrc
            rounded = ((cursor + 127) >> 7) << 7
            trips = (rounded + 2047) >> 11
            rbase = w * RCAP

            def dout(c, _, b=b):
                pltpu.sync_copy(lsrc.at[pl.ds(c * 2048, 2048)],
                                bsrc_h.at[b, pl.ds(rbase + c * 2048, 2048)])
                pltpu.sync_copy(ldst.at[pl.ds(c * 2048, 2048)],
                                bdst_h.at[b, pl.ds(rbase + c * 2048, 2048)])
                return 0

            lax.fori_loop(0, trips, dout, 0)
            cvec = jnp.where(lane == b, _splat(cursor), cvec)
        cntv[...] = cvec
        pltpu.sync_copy(cntv, cnt_h.at[w])

    return bink(src_pad, dst_pad)


def _edge_call(D, bsrc, bdst, cnt, wh, a_s, a_d, u16):
    nvreg = D // 16
    fdim = D // NHEAD
    head_of = [(k * 16) // fdim for k in range(nvreg)]
    CE = 64

    scr = dict(
        cntv=pltpu.VMEM((NW, 16), jnp.int32),
        uv=pltpu.VMEM((16,), jnp.float32),
        zbuf=pltpu.VMEM((64, D), jnp.float32),
        zden=pltpu.VMEM((64, 16), jnp.float32),
        fb=pltpu.VMEM((64, D), jnp.float32),
        dnb=pltpu.VMEM((64, 16), jnp.float32),
        acc_sh=pltpu.VMEM_SHARED((BSZ, D), jnp.float32),
        den_sh=pltpu.VMEM_SHARED((BSZ, 16), jnp.float32),
    )
    for p in (0, 1):
        scr.update({
            f"srcb{p}": pltpu.VMEM((CE,), jnp.int32),
            f"dstb{p}": pltpu.VMEM((CE,), jnp.int32),
            f"locb{p}": pltpu.VMEM((CE,), jnp.int32),
            f"asb{p}": pltpu.VMEM((CE, 16), jnp.float32),
            f"adb{p}": pltpu.VMEM((CE, 16), jnp.float32),
            f"mb{p}": pltpu.VMEM((CE, 16), jnp.float32),
            f"rows{p}": pltpu.VMEM((CE, D), jnp.float32),
            f"sga{p}": pltpu.SemaphoreType.DMA,
            f"sgd{p}": pltpu.SemaphoreType.DMA,
            f"sgw{p}": pltpu.SemaphoreType.DMA,
            f"ssm{p}": pltpu.SemaphoreType.DMA,
            f"ssr{p}": pltpu.SemaphoreType.DMA,
        })

    @functools.partial(
        pl.kernel,
        out_type=jax.ShapeDtypeStruct((NP, D), jnp.float32),
        mesh=_sc_mesh(),
        compiler_params=_SC_PARAMS,
        scratch_types=scr,
    )
    def edgek(bsrc_h, bdst_h, cnt_h, wh_h, as_h, ad_h, u_h, hout_h, **s):
        cid = lax.axis_index("c")
        sid = lax.axis_index("s")
        lane = lax.iota(jnp.int32, 16)
        zero16 = jnp.zeros((16,), jnp.float32)
        hidx = [jnp.full((16,), h, jnp.int32) for h in range(NHEAD)]
        cntv, uv = s["cntv"], s["uv"]
        zbuf, zden, fb, dnb = s["zbuf"], s["zden"], s["fb"], s["dnb"]
        acc_sh, den_sh = s["acc_sh"], s["den_sh"]
        bufs = [
            tuple(s[f"{n}{p}"] for n in
                  ("srcb", "dstb", "locb", "asb", "adb", "mb", "rows",
                   "sga", "sgd", "sgw", "ssm", "ssr"))
            for p in (0, 1)
        ]
        pltpu.sync_copy(cnt_h, cntv)
        pltpu.sync_copy(u_h, uv)
        uvv = uv[pl.ds(0, 16)]

        def zrow(r, _):
            for k in range(nvreg):
                zbuf[r, pl.ds(k * 16, 16)] = zero16
            zden[r, pl.ds(0, 16)] = zero16
            return 0

        lax.fori_loop(0, 64, zrow, 0)

        def slot_body(slot, _):
            b = slot * 2 + cid

            @pl.when(b < NBKT)
            def _process():
                def zc(i, _):
                    pltpu.sync_copy(zbuf, acc_sh.at[pl.ds(sid * 256 + i * 64,
                                                          64)])
                    pltpu.sync_copy(zden, den_sh.at[pl.ds(sid * 256 + i * 64,
                                                          64)])
                    return 0

                lax.fori_loop(0, 4, zc, 0)
                plsc.subcore_barrier()

                for wo in range(2):
                    w = sid * 2 + wo
                    crow = cntv[w, pl.ds(0, 16)]
                    cnt_wb = jnp.take(crow, _splat(b))[0]
                    trips = (cnt_wb + (CE - 1)) >> 6
                    pairs = (trips + 1) >> 1

                    def pair(t, _, b=b, w=w, cnt_wb=cnt_wb, trips=trips):
                        gath = [None, None]
                        scat = [None, None]
                        for half in (0, 1):
                            c = 2 * t + half
                            (srcb, dstb, locb, asb, adb, mb, rows,
                             sga, sgd, sgw, ssm, ssr) = bufs[half]

                            @pl.when(c < trips)
                            def _issue(c=c, srcb=srcb, dstb=dstb, half=half):
                                off = w * RCAP + c * CE
                                pltpu.sync_copy(bsrc_h.at[b, pl.ds(off, CE)],
                                                srcb)
                                pltpu.sync_copy(bdst_h.at[b, pl.ds(off, CE)],
                                                dstb)

                            @pl.when(c < trips)
                            def _gath(c=c, half=half):
                                cp3 = pltpu.async_copy(wh_h.at[srcb], rows,
                                                       sgw)
                                cp1 = pltpu.async_copy(as_h.at[srcb], asb, sga)
                                cp2 = pltpu.async_copy(ad_h.at[dstb], adb, sgd)

                        for half in (0, 1):
                            c = 2 * t + half
                            (srcb, dstb, locb, asb, adb, mb, rows,
                             sga, sgd, sgw, ssm, ssr) = bufs[half]

                            @pl.when(c < trips)
                            def _compute(c=c, srcb=srcb, dstb=dstb, locb=locb,
                                         asb=asb, adb=adb, mb=mb, rows=rows,
                                         sga=sga, sgd=sgd, sgw=sgw, ssm=ssm,
                                         ssr=ssr):
                                @plsc.parallel_loop(0, CE // 16, unroll=2)
                                def lg(g):
                                    dv = dstb[pl.ds(g * 16, 16)]
                                    locb[pl.ds(g * 16, 16)] = dv & (BSZ - 1)

                                pltpu.make_async_copy(as_h.at[srcb], asb,
                                                      sga).wait()
                                pltpu.make_async_copy(ad_h.at[dstb], adb,
                                                      sgd).wait()
                                ebase = c * CE

                                @pl.when(c < trips - 1)
                                def _m_full():
                                    @plsc.parallel_loop(0, CE, unroll=4)
                                    def edge_m(e):
                                        asv = asb[e, pl.ds(0, 16)]
                                        adv = adb[e, pl.ds(0, 16)]
                                        sv = asv + adv
                                        ev = jnp.maximum(sv, 0.2 * sv) - uvv
                                        mb[e, pl.ds(0, 16)] = jnp.exp(ev)

                                @pl.when(c == trips - 1)
                                def _m_masked():
                                    @plsc.parallel_loop(0, CE, unroll=4)
                                    def edge_m(e):
                                        asv = asb[e, pl.ds(0, 16)]
                                        adv = adb[e, pl.ds(0, 16)]
                                        sv = asv + adv
                                        ev = jnp.maximum(sv, 0.2 * sv) - uvv
                                        m = jnp.exp(ev)
                                        valid = (_splat(ebase + e)
                                                 < _splat(cnt_wb))
                                        m = jnp.where(valid, m, 0.0)
                                        mb[e, pl.ds(0, 16)] = m

                                pltpu.make_async_copy(wh_h.at[srcb], rows,
                                                      sgw).wait()

                                @plsc.parallel_loop(0, CE, unroll=2)
                                def edge_scale(e):
                                    m = mb[e, pl.ds(0, 16)]
                                    sps = [jnp.take(m, hidx[h])
                                           for h in range(NHEAD)]
                                    for k in range(nvreg):
                                        r = rows[e, pl.ds(k * 16, 16)]
                                        rows[e, pl.ds(k * 16, 16)] = (
                                            r * sps[head_of[k]])

                                pltpu.async_copy(mb, den_sh.at[locb], ssm,
                                                 add=True)
                                pltpu.async_copy(rows, acc_sh.at[locb], ssr,
                                                 add=True)

                        for half in (0, 1):
                            c = 2 * t + half
                            (srcb, dstb, locb, asb, adb, mb, rows,
                             sga, sgd, sgw, ssm, ssr) = bufs[half]

                            @pl.when(c < trips)
                            def _drain(mb=mb, rows=rows, locb=locb, ssm=ssm,
                                       ssr=ssr):
                                pltpu.make_async_copy(
                                    mb, den_sh.at[locb], ssm).wait()
                                pltpu.make_async_copy(
                                    rows, acc_sh.at[locb], ssr).wait()
                        return 0

                    lax.fori_loop(0, pairs, pair, 0)
                plsc.subcore_barrier()

                def fin(i, _):
                    r0 = sid * 256 + i * 64
                    pltpu.sync_copy(acc_sh.at[pl.ds(r0, 64)], fb)
                    pltpu.sync_copy(den_sh.at[pl.ds(r0, 64)], dnb)

                    @plsc.parallel_loop(0, 64, unroll=2)
                    def frow(n):
                        dv = dnb[n, pl.ds(0, 16)]
                        rec = 1.0 / (dv + 1e-16)
                        sps = [jnp.take(rec, hidx[h]) for h in range(NHEAD)]
                        for k in range(nvreg):
                            x = fb[n, pl.ds(k * 16, 16)] * sps[head_of[k]]
                            y = jnp.where(x > 0, x, jnp.exp(x) - 1.0)
                            fb[n, pl.ds(k * 16, 16)] = y
                    pltpu.sync_copy(fb, hout_h.at[pl.ds(b * BSZ + r0, 64)])
                    return 0

                lax.fori_loop(0, 4, fin, 0)
                plsc.subcore_barrier()

            return 0

        lax.fori_loop(0, (NBKT + 1) // 2, slot_body, 0)

    return edgek(bsrc, bdst, cnt, wh, a_s, a_d, u16)


def _prep_call(x_pad, w_pad, ase, ade, D):
    kdim = x_pad.shape[1]
    grid = NP // ROWBLK

    def prep(x_ref, w_ref, ase_ref, ade_ref, wh_ref, as_ref, ad_ref):
        xb = x_ref[...]
        whb = jnp.dot(xb, w_ref[...], preferred_element_type=jnp.float32)
        wh_ref[...] = whb
        as_ref[...] = jnp.dot(whb, ase_ref[...],
                              preferred_element_type=jnp.float32)
        ad_ref[...] = jnp.dot(whb, ade_ref[...],
                              preferred_element_type=jnp.float32)

    return pl.pallas_call(
        prep,
        grid=(grid,),
        in_specs=[
            pl.BlockSpec((ROWBLK, kdim), lambda i: (i, 0)),
            pl.BlockSpec((kdim, D), lambda i: (0, 0)),
            pl.BlockSpec((D, 16), lambda i: (0, 0)),
            pl.BlockSpec((D, 16), lambda i: (0, 0)),
        ],
        out_specs=[
            pl.BlockSpec((ROWBLK, D), lambda i: (i, 0)),
            pl.BlockSpec((ROWBLK, 16), lambda i: (i, 0)),
            pl.BlockSpec((ROWBLK, 16), lambda i: (i, 0)),
        ],
        out_shape=[
            jax.ShapeDtypeStruct((NP, D), jnp.float32),
            jax.ShapeDtypeStruct((NP, 16), jnp.float32),
            jax.ShapeDtypeStruct((NP, 16), jnp.float32),
        ],
    )(x_pad, w_pad, ase, ade)


def _u_call(a_s, a_d):
    def uk(as_ref, ad_ref, u_ref):
        u = (jnp.max(as_ref[...], axis=0, keepdims=True)
             + jnp.max(ad_ref[...], axis=0, keepdims=True))
        u_ref[...] = jnp.maximum(u, 0.2 * u)

    return pl.pallas_call(
        uk,
        out_shape=jax.ShapeDtypeStruct((1, 16), jnp.float32),
    )(a_s, a_d)


def _pool_call(h_i, h_n):
    grid = NP // ROWBLK

    def poolk(hi_ref, hn_ref, o_ref):
        @pl.when(pl.program_id(0) == 0)
        def _init():
            o_ref[...] = jnp.zeros_like(o_ref)

        s1 = jnp.sum(hi_ref[...], axis=0, keepdims=True)
        s2 = jnp.sum(hn_ref[...], axis=0, keepdims=True)
        o_ref[...] += jnp.concatenate([s1, s2], axis=1)

    return pl.pallas_call(
        poolk,
        grid=(grid,),
        in_specs=[
            pl.BlockSpec((ROWBLK, D2), lambda i: (i, 0)),
            pl.BlockSpec((ROWBLK, D2), lambda i: (i, 0)),
        ],
        out_specs=pl.BlockSpec((1, 2 * D2), lambda i: (0, 0)),
        out_shape=jax.ShapeDtypeStruct((1, 2 * D2), jnp.float32),
    )(h_i, h_n)


def _tail_call(pooled, Wd, bd):
    def tailk(x_ref, wd_ref, bd_ref, o_ref):
        x = x_ref[...]
        nrm = jnp.maximum(jnp.sqrt(jnp.sum(x * x)), 1e-12)
        o_ref[...] = (jnp.sum(x * wd_ref[...], axis=1, keepdims=True) / nrm
                      + bd_ref[...])

    return pl.pallas_call(
        tailk,
        out_shape=jax.ShapeDtypeStruct((1, 1), jnp.float32),
    )(pooled, Wd.reshape(1, -1), bd.reshape(1, 1))


def _expand_alpha(a):
    hh, f = a.shape
    d = hh * f
    cols = jnp.arange(16)[None, :]
    rowh = (jnp.arange(d) // f)[:, None]
    vals = a.reshape(d, 1)
    return jnp.where((cols == rowh) | (cols == rowh + 8), vals, 0.0)


def _pad_edges(ei):
    src = ei[0]
    dst = ei[1]
    pad = EPAD - E
    src = jnp.concatenate([src, jnp.zeros((pad,), jnp.int32)])
    dst = jnp.concatenate([dst, jnp.full((pad,), 65535, jnp.int32)])
    return src, dst


def kernel(node_feats, edge_index_int, edge_index_nh, W1, a1_src, a1_dst,
           W2, a2_src, a2_dst, Wd, bd):
    f32 = jnp.float32
    x1 = jnp.zeros((NP, 128), f32).at[:N, :11].set(node_feats)
    W1p = jnp.zeros((128, D1), f32).at[:11, :].set(W1)
    W2p = jnp.zeros((128, D2), f32).at[:D1, :].set(W2)
    ase1 = _expand_alpha(a1_src)
    ade1 = _expand_alpha(a1_dst)
    ase2 = _expand_alpha(a2_src)
    ade2 = _expand_alpha(a2_dst)

    si, di = _pad_edges(edge_index_int)
    sn, dn = _pad_edges(edge_index_nh)
    bs_i, bd_i, cnt_i = _bin_call(si, di)
    bs_n, bd_n, cnt_n = _bin_call(sn, dn)

    wh1, as1, ad1 = _prep_call(x1, W1p, ase1, ade1, D1)
    u1 = _u_call(as1, ad1).reshape(16)
    h1_i = _edge_call(D1, bs_i, bd_i, cnt_i, wh1, as1, ad1, u1)
    h1_n = _edge_call(D1, bs_n, bd_n, cnt_n, wh1, as1, ad1, u1)

    x2_i = jnp.pad(h1_i, ((0, 0), (0, 128 - D1)))
    wh2_i, as2_i, ad2_i = _prep_call(x2_i, W2p, ase2, ade2, D2)
    u2_i = _u_call(as2_i, ad2_i).reshape(16)
    h2_i = _edge_call(D2, bs_i, bd_i, cnt_i, wh2_i, as2_i, ad2_i, u2_i)

    x2_n = jnp.pad(h1_n, ((0, 0), (0, 128 - D1)))
    wh2_n, as2_n, ad2_n = _prep_call(x2_n, W2p, ase2, ade2, D2)
    u2_n = _u_call(as2_n, ad2_n).reshape(16)
    h2_n = _edge_call(D2, bs_n, bd_n, cnt_n, wh2_n, as2_n, ad2_n, u2_n)

    pooled = _pool_call(h2_i, h2_n)
    out = _tail_call(pooled, Wd, bd)
    return jnp.squeeze(out, 1)


# speedup vs baseline: 115.1360x; 1.1142x over previous
import functools

import jax
import jax.numpy as jnp
from jax import lax
from jax.experimental import pallas as pl
from jax.experimental.pallas import tpu as pltpu
from jax.experimental.pallas import tpu_sc as plsc

N = 50000
E = 800000
NP = 53248
BSZ = 4096
NBKT = 13
BSH = 12
NW = 32
EPW = 25600
EPAD = NW * EPW
RCAP = 28800
CH = 128
NHEAD = 6
D1, D2 = 96, 192
ROWBLK = 512


def _splat(s):
    return lax.broadcast_in_dim(s, (16,), ())


@functools.lru_cache(maxsize=None)
def _sc_mesh():
    return plsc.VectorSubcoreMesh(core_axis_name="c", subcore_axis_name="s")


_SC_PARAMS = pltpu.CompilerParams(needs_layout_passes=False,
                                  use_tc_tiling_on_sc=False)


def _bin_call(src_pad, dst_pad):
    @functools.partial(
        pl.kernel,
        out_type=(
            jax.ShapeDtypeStruct((NBKT + 1, NW * RCAP), jnp.int32),
            jax.ShapeDtypeStruct((NBKT + 1, NW * RCAP), jnp.int32),
            jax.ShapeDtypeStruct((NW, 16), jnp.int32),
        ),
        mesh=_sc_mesh(),
        compiler_params=_SC_PARAMS,
        scratch_types=dict(
            srcv=pltpu.VMEM((EPW,), jnp.int32),
            dstv=pltpu.VMEM((EPW,), jnp.int32),
            lsrc=pltpu.VMEM((RCAP,), jnp.int32),
            ldst=pltpu.VMEM((RCAP,), jnp.int32),
            cntv=pltpu.VMEM((16,), jnp.int32),
        ),
    )
    def bink(src_h, dst_h, bsrc_h, bdst_h, cnt_h, srcv, dstv, lsrc, ldst,
             cntv):
        cid = lax.axis_index("c")
        sid = lax.axis_index("s")
        w = sid * 2 + cid
        lane = lax.iota(jnp.int32, 16)
        base = w * EPW
        for c in range(8):
            pltpu.sync_copy(src_h.at[pl.ds(base + c * 3200, 3200)],
                            srcv.at[pl.ds(c * 3200, 3200)])
            pltpu.sync_copy(dst_h.at[pl.ds(base + c * 3200, 3200)],
                            dstv.at[pl.ds(c * 3200, 3200)])
        cvec = jnp.zeros((16,), jnp.int32)
        for b in range(NBKT):
            def grp(g, cur, b=b):
                dv = dstv[pl.ds(g * 16, 16)]
                sv = srcv[pl.ds(g * 16, 16)]
                mask = (dv >> BSH) == b
                sk, svv, _ = plsc.sort_key_val(dv, sv, mask=mask)
                ldst[pl.ds(cur, 16)] = sk
                lsrc[pl.ds(cur, 16)] = svv
                pc = plsc.all_reduce_population_count(mask)
                return cur + pc[0]

            cursor = lax.fori_loop(0, EPW // 16, grp, jnp.int32(0))
            pdst = _splat(jnp.int32(b * BSZ)) + lane
            psrc = lane
            for pg in range(8):
                ldst[pl.ds(cursor + pg * 16, 16)] = pdst
                lsrc[pl.ds(cursor + pg * 16, 16)] = psrc
            rounded = ((cursor + 255) >> 7) << 7
            trips = (rounded + 2047) >> 11
            rbase = w * RCAP

            def dout(c, _, b=b):
                pltpu.sync_copy(lsrc.at[pl.ds(c * 2048, 2048)],
                                bsrc_h.at[b, pl.ds(rbase + c * 2048, 2048)])
                pltpu.sync_copy(ldst.at[pl.ds(c * 2048, 2048)],
                                bdst_h.at[b, pl.ds(rbase + c * 2048, 2048)])
                return 0

            lax.fori_loop(0, trips, dout, 0)
            cvec = jnp.where(lane == b, _splat(cursor), cvec)
        cntv[...] = cvec
        pltpu.sync_copy(cntv, cnt_h.at[w])

    return bink(src_pad, dst_pad)


def _edge_call(D, bsrc, bdst, cnt, wh, a_s, a_d, u16):
    nvreg = D // 16
    fdim = D // NHEAD
    head_of = [(k * 16) // fdim for k in range(nvreg)]
    CE = 96

    scr = dict(
        cntv=pltpu.VMEM((NW, 16), jnp.int32),
        uv=pltpu.VMEM((16,), jnp.float32),
        zbuf=pltpu.VMEM((64, D), jnp.float32),
        zden=pltpu.VMEM((64, 16), jnp.float32),
        fb=pltpu.VMEM((64, D), jnp.float32),
        dnb=pltpu.VMEM((64, 16), jnp.float32),
        acc_sh=pltpu.VMEM_SHARED((BSZ, D), jnp.float32),
        den_sh=pltpu.VMEM_SHARED((BSZ, 16), jnp.float32),
    )
    for p in (0, 1):
        scr.update({
            f"srcb{p}": pltpu.VMEM((CE,), jnp.int32),
            f"dstb{p}": pltpu.VMEM((CE,), jnp.int32),
            f"locb{p}": pltpu.VMEM((CE,), jnp.int32),
            f"asb{p}": pltpu.VMEM((CE, 16), jnp.float32),
            f"adb{p}": pltpu.VMEM((CE, 16), jnp.float32),
            f"mb{p}": pltpu.VMEM((CE, 16), jnp.float32),
            f"rows{p}": pltpu.VMEM((CE, D), jnp.float32),
            f"sga{p}": pltpu.SemaphoreType.DMA,
            f"sgd{p}": pltpu.SemaphoreType.DMA,
            f"sgw{p}": pltpu.SemaphoreType.DMA,
            f"ssm{p}": pltpu.SemaphoreType.DMA,
            f"ssr{p}": pltpu.SemaphoreType.DMA,
        })

    @functools.partial(
        pl.kernel,
        out_type=jax.ShapeDtypeStruct((NP, D), jnp.float32),
        mesh=_sc_mesh(),
        compiler_params=_SC_PARAMS,
        scratch_types=scr,
    )
    def edgek(bsrc_h, bdst_h, cnt_h, wh_h, as_h, ad_h, u_h, hout_h, **s):
        cid = lax.axis_index("c")
        sid = lax.axis_index("s")
        lane = lax.iota(jnp.int32, 16)
        zero16 = jnp.zeros((16,), jnp.float32)
        hidx = [jnp.full((16,), h, jnp.int32) for h in range(NHEAD)]
        cntv, uv = s["cntv"], s["uv"]
        zbuf, zden, fb, dnb = s["zbuf"], s["zden"], s["fb"], s["dnb"]
        acc_sh, den_sh = s["acc_sh"], s["den_sh"]
        bufs = [
            tuple(s[f"{n}{p}"] for n in
                  ("srcb", "dstb", "locb", "asb", "adb", "mb", "rows",
                   "sga", "sgd", "sgw", "ssm", "ssr"))
            for p in (0, 1)
        ]
        pltpu.sync_copy(cnt_h, cntv)
        pltpu.sync_copy(u_h, uv)
        uvv = uv[pl.ds(0, 16)]

        def zrow(r, _):
            for k in range(nvreg):
                zbuf[r, pl.ds(k * 16, 16)] = zero16
            zden[r, pl.ds(0, 16)] = zero16
            return 0

        lax.fori_loop(0, 64, zrow, 0)

        def slot_body(slot, _):
            b = slot * 2 + cid

            @pl.when(b < NBKT)
            def _process():
                def zc(i, _):
                    pltpu.sync_copy(zbuf, acc_sh.at[pl.ds(sid * 256 + i * 64,
                                                          64)])
                    pltpu.sync_copy(zden, den_sh.at[pl.ds(sid * 256 + i * 64,
                                                          64)])
                    return 0

                lax.fori_loop(0, 4, zc, 0)
                plsc.subcore_barrier()

                for wo in range(2):
                    w = sid * 2 + wo
                    crow = cntv[w, pl.ds(0, 16)]
                    cnt_wb = jnp.take(crow, _splat(b))[0]
                    trips = (cnt_wb + (CE - 1)) // CE
                    pairs = (trips + 1) >> 1

                    def pair(t, _, b=b, w=w, cnt_wb=cnt_wb, trips=trips):
                        gath = [None, None]
                        scat = [None, None]
                        for half in (0, 1):
                            c = 2 * t + half
                            (srcb, dstb, locb, asb, adb, mb, rows,
                             sga, sgd, sgw, ssm, ssr) = bufs[half]

                            @pl.when(c < trips)
                            def _issue(c=c, srcb=srcb, dstb=dstb, half=half):
                                off = w * RCAP + c * CE
                                pltpu.sync_copy(bsrc_h.at[b, pl.ds(off, CE)],
                                                srcb)
                                pltpu.sync_copy(bdst_h.at[b, pl.ds(off, CE)],
                                                dstb)

                            @pl.when(c < trips)
                            def _gath(c=c, half=half):
                                cp3 = pltpu.async_copy(wh_h.at[srcb], rows,
                                                       sgw)
                                cp1 = pltpu.async_copy(as_h.at[srcb], asb, sga)
                                cp2 = pltpu.async_copy(ad_h.at[dstb], adb, sgd)

                        for half in (0, 1):
                            c = 2 * t + half
                            (srcb, dstb, locb, asb, adb, mb, rows,
                             sga, sgd, sgw, ssm, ssr) = bufs[half]

                            @pl.when(c < trips)
                            def _compute(c=c, srcb=srcb, dstb=dstb, locb=locb,
                                         asb=asb, adb=adb, mb=mb, rows=rows,
                                         sga=sga, sgd=sgd, sgw=sgw, ssm=ssm,
                                         ssr=ssr):
                                @plsc.parallel_loop(0, CE // 16, unroll=2)
                                def lg(g):
                                    dv = dstb[pl.ds(g * 16, 16)]
                                    locb[pl.ds(g * 16, 16)] = dv & (BSZ - 1)

                                pltpu.make_async_copy(as_h.at[srcb], asb,
                                                      sga).wait()
                                pltpu.make_async_copy(ad_h.at[dstb], adb,
                                                      sgd).wait()
                                ebase = c * CE

                                @pl.when(c < trips - 1)
                                def _m_full():
                                    @plsc.parallel_loop(0, CE, unroll=4)
                                    def edge_m(e):
                                        asv = asb[e, pl.ds(0, 16)]
                                        adv = adb[e, pl.ds(0, 16)]
                                        sv = asv + adv
                                        ev = jnp.maximum(sv, 0.2 * sv) - uvv
                                        mb[e, pl.ds(0, 16)] = jnp.exp(ev)

                                @pl.when(c == trips - 1)
                                def _m_masked():
                                    @plsc.parallel_loop(0, CE, unroll=4)
                                    def edge_m(e):
                                        asv = asb[e, pl.ds(0, 16)]
                                        adv = adb[e, pl.ds(0, 16)]
                                        sv = asv + adv
                                        ev = jnp.maximum(sv, 0.2 * sv) - uvv
                                        m = jnp.exp(ev)
                                        valid = (_splat(ebase + e)
                                                 < _splat(cnt_wb))
                                        m = jnp.where(valid, m, 0.0)
                                        mb[e, pl.ds(0, 16)] = m

                                pltpu.make_async_copy(wh_h.at[srcb], rows,
                                                      sgw).wait()

                                @plsc.parallel_loop(0, CE, unroll=4)
                                def edge_scale(e):
                                    m = mb[e, pl.ds(0, 16)]
                                    sps = [jnp.take(m, hidx[h])
                                           for h in range(NHEAD)]
                                    for k in range(nvreg):
                                        r = rows[e, pl.ds(k * 16, 16)]
                                        rows[e, pl.ds(k * 16, 16)] = (
                                            r * sps[head_of[k]])

                                pltpu.async_copy(mb, den_sh.at[locb], ssm,
                                                 add=True)
                                pltpu.async_copy(rows, acc_sh.at[locb], ssr,
                                                 add=True)

                        for half in (0, 1):
                            c = 2 * t + half
                            (srcb, dstb, locb, asb, adb, mb, rows,
                             sga, sgd, sgw, ssm, ssr) = bufs[half]

                            @pl.when(c < trips)
                            def _drain(mb=mb, rows=rows, locb=locb, ssm=ssm,
                                       ssr=ssr):
                                pltpu.make_async_copy(
                                    mb, den_sh.at[locb], ssm).wait()
                                pltpu.make_async_copy(
                                    rows, acc_sh.at[locb], ssr).wait()
                        return 0

                    lax.fori_loop(0, pairs, pair, 0)
                plsc.subcore_barrier()

                def fin(i, _):
                    r0 = sid * 256 + i * 64
                    pltpu.sync_copy(acc_sh.at[pl.ds(r0, 64)], fb)
                    pltpu.sync_copy(den_sh.at[pl.ds(r0, 64)], dnb)

                    @plsc.parallel_loop(0, 64, unroll=2)
                    def frow(n):
                        dv = dnb[n, pl.ds(0, 16)]
                        rec = 1.0 / (dv + 1e-16)
                        sps = [jnp.take(rec, hidx[h]) for h in range(NHEAD)]
                        for k in range(nvreg):
                            x = fb[n, pl.ds(k * 16, 16)] * sps[head_of[k]]
                            y = jnp.where(x > 0, x, jnp.exp(x) - 1.0)
                            fb[n, pl.ds(k * 16, 16)] = y
                    pltpu.sync_copy(fb, hout_h.at[pl.ds(b * BSZ + r0, 64)])
                    return 0

                lax.fori_loop(0, 4, fin, 0)
                plsc.subcore_barrier()

            return 0

        lax.fori_loop(0, (NBKT + 1) // 2, slot_body, 0)

    return edgek(bsrc, bdst, cnt, wh, a_s, a_d, u16)


def _prep_call(x_pad, w_pad, ase, ade, D):
    kdim = x_pad.shape[1]
    grid = NP // ROWBLK

    def prep(x_ref, w_ref, ase_ref, ade_ref, wh_ref, as_ref, ad_ref):
        xb = x_ref[...]
        whb = jnp.dot(xb, w_ref[...], preferred_element_type=jnp.float32)
        wh_ref[...] = whb
        as_ref[...] = jnp.dot(whb, ase_ref[...],
                              preferred_element_type=jnp.float32)
        ad_ref[...] = jnp.dot(whb, ade_ref[...],
                              preferred_element_type=jnp.float32)

    return pl.pallas_call(
        prep,
        grid=(grid,),
        in_specs=[
            pl.BlockSpec((ROWBLK, kdim), lambda i: (i, 0)),
            pl.BlockSpec((kdim, D), lambda i: (0, 0)),
            pl.BlockSpec((D, 16), lambda i: (0, 0)),
            pl.BlockSpec((D, 16), lambda i: (0, 0)),
        ],
        out_specs=[
            pl.BlockSpec((ROWBLK, D), lambda i: (i, 0)),
            pl.BlockSpec((ROWBLK, 16), lambda i: (i, 0)),
            pl.BlockSpec((ROWBLK, 16), lambda i: (i, 0)),
        ],
        out_shape=[
            jax.ShapeDtypeStruct((NP, D), jnp.float32),
            jax.ShapeDtypeStruct((NP, 16), jnp.float32),
            jax.ShapeDtypeStruct((NP, 16), jnp.float32),
        ],
    )(x_pad, w_pad, ase, ade)


def _u_call(a_s, a_d):
    def uk(as_ref, ad_ref, u_ref):
        u = (jnp.max(as_ref[...], axis=0, keepdims=True)
             + jnp.max(ad_ref[...], axis=0, keepdims=True))
        u_ref[...] = jnp.maximum(u, 0.2 * u)

    return pl.pallas_call(
        uk,
        out_shape=jax.ShapeDtypeStruct((1, 16), jnp.float32),
    )(a_s, a_d)


def _pool_call(h_i, h_n):
    grid = NP // ROWBLK

    def poolk(hi_ref, hn_ref, o_ref):
        @pl.when(pl.program_id(0) == 0)
        def _init():
            o_ref[...] = jnp.zeros_like(o_ref)

        s1 = jnp.sum(hi_ref[...], axis=0, keepdims=True)
        s2 = jnp.sum(hn_ref[...], axis=0, keepdims=True)
        o_ref[...] += jnp.concatenate([s1, s2], axis=1)

    return pl.pallas_call(
        poolk,
        grid=(grid,),
        in_specs=[
            pl.BlockSpec((ROWBLK, D2), lambda i: (i, 0)),
            pl.BlockSpec((ROWBLK, D2), lambda i: (i, 0)),
        ],
        out_specs=pl.BlockSpec((1, 2 * D2), lambda i: (0, 0)),
        out_shape=jax.ShapeDtypeStruct((1, 2 * D2), jnp.float32),
    )(h_i, h_n)


def _tail_call(pooled, Wd, bd):
    def tailk(x_ref, wd_ref, bd_ref, o_ref):
        x = x_ref[...]
        nrm = jnp.maximum(jnp.sqrt(jnp.sum(x * x)), 1e-12)
        o_ref[...] = (jnp.sum(x * wd_ref[...], axis=1, keepdims=True) / nrm
                      + bd_ref[...])

    return pl.pallas_call(
        tailk,
        out_shape=jax.ShapeDtypeStruct((1, 1), jnp.float32),
    )(pooled, Wd.reshape(1, -1), bd.reshape(1, 1))


def _expand_alpha(a):
    hh, f = a.shape
    d = hh * f
    cols = jnp.arange(16)[None, :]
    rowh = (jnp.arange(d) // f)[:, None]
    vals = a.reshape(d, 1)
    return jnp.where((cols == rowh) | (cols == rowh + 8), vals, 0.0)


def _pad_edges(ei):
    src = ei[0]
    dst = ei[1]
    pad = EPAD - E
    src = jnp.concatenate([src, jnp.zeros((pad,), jnp.int32)])
    dst = jnp.concatenate([dst, jnp.full((pad,), 65535, jnp.int32)])
    return src, dst


def kernel(node_feats, edge_index_int, edge_index_nh, W1, a1_src, a1_dst,
           W2, a2_src, a2_dst, Wd, bd):
    f32 = jnp.float32
    x1 = jnp.zeros((NP, 128), f32).at[:N, :11].set(node_feats)
    W1p = jnp.zeros((128, D1), f32).at[:11, :].set(W1)
    W2p = jnp.zeros((128, D2), f32).at[:D1, :].set(W2)
    ase1 = _expand_alpha(a1_src)
    ade1 = _expand_alpha(a1_dst)
    ase2 = _expand_alpha(a2_src)
    ade2 = _expand_alpha(a2_dst)

    si, di = _pad_edges(edge_index_int)
    sn, dn = _pad_edges(edge_index_nh)
    bs_i, bd_i, cnt_i = _bin_call(si, di)
    bs_n, bd_n, cnt_n = _bin_call(sn, dn)

    wh1, as1, ad1 = _prep_call(x1, W1p, ase1, ade1, D1)
    u1 = _u_call(as1, ad1).reshape(16)
    h1_i = _edge_call(D1, bs_i, bd_i, cnt_i, wh1, as1, ad1, u1)
    h1_n = _edge_call(D1, bs_n, bd_n, cnt_n, wh1, as1, ad1, u1)

    x2_i = jnp.pad(h1_i, ((0, 0), (0, 128 - D1)))
    wh2_i, as2_i, ad2_i = _prep_call(x2_i, W2p, ase2, ade2, D2)
    u2_i = _u_call(as2_i, ad2_i).reshape(16)
    h2_i = _edge_call(D2, bs_i, bd_i, cnt_i, wh2_i, as2_i, ad2_i, u2_i)

    x2_n = jnp.pad(h1_n, ((0, 0), (0, 128 - D1)))
    wh2_n, as2_n, ad2_n = _prep_call(x2_n, W2p, ase2, ade2, D2)
    u2_n = _u_call(as2_n, ad2_n).reshape(16)
    h2_n = _edge_call(D2, bs_n, bd_n, cnt_n, wh2_n, as2_n, ad2_n, u2_n)

    pooled = _pool_call(h2_i, h2_n)
    out = _tail_call(pooled, Wd, bd)
    return jnp.squeeze(out, 1)
